# baseline (speedup 1.0000x reference)
import functools

import jax
import jax.numpy as jnp
from jax import lax
from jax.experimental import pallas as pl
from jax.experimental.pallas import tpu as pltpu
from jax.experimental.pallas import tpu_sc as plsc

EMBED_DIM = 64
HIDDEN_DIM = 256
VOCAB_SIZE = 1000000
CATEGORY_SIZE = 1000
CAT_PAD = 1024
BATCH = 16384
VOCAB_TRUNC = 999936
TAIL = VOCAB_SIZE - VOCAB_TRUNC

NUM_CORES = 2
NUM_SUBCORES = 16
NUM_WORKERS = NUM_CORES * NUM_SUBCORES
B_PER_W = BATCH // NUM_WORKERS
LANES = 16
RING = 16
ROUNDS = B_PER_W // RING


def _sc_title_body(idx_hbm, tab_hbm, out_hbm, idx_v, eidx_v, rows_v, sem):
    wid = lax.axis_index("s") * NUM_CORES + lax.axis_index("c")
    base = wid * B_PER_W
    pltpu.sync_copy(idx_hbm.at[pl.ds(base, B_PER_W)], idx_v)
    iota = lax.iota(jnp.int32, LANES)

    def round_body(r, carry):
        copies = []
        for s in range(RING):
            e = r * RING + s
            v_splat = plsc.load_gather(idx_v, [jnp.full((LANES,), 0, jnp.int32) + e])
            v_cl = jnp.where(v_splat < VOCAB_TRUNC, v_splat, 0)
            for c in range(EMBED_DIM // LANES):
                eidx_v[pl.ds(s * EMBED_DIM + c * LANES, LANES)] = (
                    (iota + c * LANES) * VOCAB_TRUNC + v_cl)
            copies.append(pltpu.async_copy(
                tab_hbm.at[eidx_v.at[pl.ds(s * EMBED_DIM, EMBED_DIM)]],
                rows_v.at[pl.ds(e * EMBED_DIM, EMBED_DIM)], sem))
        for cp in copies:
            cp.wait()
        return carry

    lax.fori_loop(0, ROUNDS, round_body, 0)
    pltpu.sync_copy(rows_v, out_hbm.at[pl.ds(base * EMBED_DIM, B_PER_W * EMBED_DIM)])


_sc_title_gather = functools.partial(
    pl.kernel,
    mesh=plsc.VectorSubcoreMesh(
        core_axis_name="c", subcore_axis_name="s",
        num_cores=NUM_CORES, num_subcores=NUM_SUBCORES),
    out_type=jax.ShapeDtypeStruct((BATCH * EMBED_DIM,), jnp.float32),
    scratch_types=[
        pltpu.VMEM((B_PER_W,), jnp.int32),
        pltpu.VMEM((RING * EMBED_DIM,), jnp.int32),
        pltpu.VMEM((B_PER_W * EMBED_DIM,), jnp.float32),
        pltpu.SemaphoreType.DMA,
    ],
    compiler_params=pltpu.CompilerParams(needs_layout_passes=False),
)(_sc_title_body)


def _depad_body(tabt_ref, flat_ref, sem):
    copies = [
        pltpu.make_async_copy(
            tabt_ref.at[d, pl.ds(0, VOCAB_TRUNC)],
            flat_ref.at[pl.ds(d * VOCAB_TRUNC, VOCAB_TRUNC)], sem)
        for d in range(EMBED_DIM)
    ]
    for cp in copies:
        cp.start()
    for cp in copies:
        cp.wait()


def _depad(tabt):
    return pl.pallas_call(
        _depad_body,
        in_specs=[pl.BlockSpec(memory_space=pltpu.MemorySpace.HBM)],
        out_specs=pl.BlockSpec(memory_space=pltpu.MemorySpace.HBM),
        out_shape=jax.ShapeDtypeStruct((VOCAB_TRUNC * EMBED_DIM,), jnp.float32),
        scratch_shapes=[pltpu.SemaphoreType.DMA],
    )(tabt)


def _mlp_body(x1_ref, tidx_ref, tail_ref, cidx_ref, ctab_ref, w1a_ref, w1b_ref,
              b1_ref, w2_ref, b2_ref, o_ref):
    m = x1_ref.shape[0]
    tv = tidx_ref[...]
    onehot_t = (jax.lax.broadcasted_iota(jnp.int32, (m, TAIL), 1)
                == (tv - VOCAB_TRUNC)).astype(jnp.float32)
    x1 = (x1_ref[...] * (tv < VOCAB_TRUNC).astype(jnp.float32)
          + jnp.dot(onehot_t, tail_ref[...], preferred_element_type=jnp.float32))
    onehot = (jax.lax.broadcasted_iota(jnp.int32, (m, CAT_PAD), 1)
              == cidx_ref[...]).astype(jnp.float32)
    x2 = jnp.dot(onehot, ctab_ref[...], preferred_element_type=jnp.float32)
    h = jnp.dot(x1, w1a_ref[...], preferred_element_type=jnp.float32)
    h = h + jnp.dot(x2, w1b_ref[...], preferred_element_type=jnp.float32)
    h = jnp.maximum(h + b1_ref[...], 0.0)
    out = jnp.dot(h, w2_ref[...], preferred_element_type=jnp.float32) + b2_ref[...]
    norm = jnp.sqrt(jnp.sum(out * out, axis=1, keepdims=True))
    o_ref[...] = out / jnp.maximum(norm, 1e-12)


def _mlp(trows, tidx, tail_rows, cidx, ctab_pad, w1a, w1b, b1, w2, b2,
         block_m=2048):
    grid = (BATCH // block_m,)
    return pl.pallas_call(
        _mlp_body,
        grid=grid,
        in_specs=[
            pl.BlockSpec((block_m, EMBED_DIM), lambda i: (i, 0)),
            pl.BlockSpec((block_m, 1), lambda i: (i, 0)),
            pl.BlockSpec((TAIL, EMBED_DIM), lambda i: (0, 0)),
            pl.BlockSpec((block_m, 1), lambda i: (i, 0)),
            pl.BlockSpec((CAT_PAD, EMBED_DIM), lambda i: (0, 0)),
            pl.BlockSpec((EMBED_DIM, HIDDEN_DIM), lambda i: (0, 0)),
            pl.BlockSpec((EMBED_DIM, HIDDEN_DIM), lambda i: (0, 0)),
            pl.BlockSpec((1, HIDDEN_DIM), lambda i: (0, 0)),
            pl.BlockSpec((HIDDEN_DIM, EMBED_DIM), lambda i: (0, 0)),
            pl.BlockSpec((1, EMBED_DIM), lambda i: (0, 0)),
        ],
        out_specs=pl.BlockSpec((block_m, EMBED_DIM), lambda i: (i, 0)),
        out_shape=jax.ShapeDtypeStruct((BATCH, EMBED_DIM), jnp.float32),
    )(trows, tidx, tail_rows, cidx, ctab_pad, w1a, w1b, b1, w2, b2)


def kernel(title_idx, category_idx, title_table, category_table, W1, b1, W2, b2):
    tidx = title_idx.astype(jnp.int32)
    tab_flat = _depad(title_table.T)
    trows_flat = _sc_title_gather(tidx, tab_flat)
    trows = trows_flat.reshape(BATCH, EMBED_DIM)
    ctab_pad = jnp.zeros((CAT_PAD, EMBED_DIM), jnp.float32).at[:CATEGORY_SIZE].set(
        category_table)
    tail_rows = title_table[VOCAB_TRUNC:]
    w1t = W1.T
    return _mlp(trows, tidx.reshape(BATCH, 1), tail_rows,
                category_idx.astype(jnp.int32).reshape(BATCH, 1),
                ctab_pad, w1t[:EMBED_DIM], w1t[EMBED_DIM:],
                b1.reshape(1, HIDDEN_DIM), W2.T, b2.reshape(1, EMBED_DIM))


# speedup vs baseline: 25.3775x; 25.3775x over previous
import functools

import jax
import jax.numpy as jnp
from jax import lax
from jax.experimental import pallas as pl
from jax.experimental.pallas import tpu as pltpu
from jax.experimental.pallas import tpu_sc as plsc

EMBED_DIM = 64
ROW_PAD = 128
HIDDEN_DIM = 256
VOCAB_SIZE = 1000000
CATEGORY_SIZE = 1000
CAT_PAD = 1024
BATCH = 16384

NUM_CORES = 2
NUM_SUBCORES = 16
NUM_WORKERS = NUM_CORES * NUM_SUBCORES
B_PER_W = BATCH // NUM_WORKERS
CHUNK = 128
CHUNKS_PER_W = B_PER_W // CHUNK

TBLK = 16384
TGRID = -(-VOCAB_SIZE // TBLK)


def _transpose_body(x_ref, eye_ref, o_ref):
    xt = jax.lax.dot_general(x_ref[...], eye_ref[...], (((0,), (0,)), ((), ())),
                             preferred_element_type=jnp.float32)
    o_ref[:, :EMBED_DIM] = xt


def _transpose_table(tabt, eye):
    return pl.pallas_call(
        _transpose_body,
        grid=(TGRID,),
        in_specs=[
            pl.BlockSpec((EMBED_DIM, TBLK), lambda i: (0, i)),
            pl.BlockSpec((EMBED_DIM, EMBED_DIM), lambda i: (0, 0)),
        ],
        out_specs=pl.BlockSpec((TBLK, ROW_PAD), lambda i: (i, 0)),
        out_shape=jax.ShapeDtypeStruct((VOCAB_SIZE, ROW_PAD), jnp.float32),
    )(tabt, eye)


def _sc_gather_body(idx_hbm, tab_hbm, out_hbm, idx_v, rows_v, sem):
    wid = lax.axis_index("s") * NUM_CORES + lax.axis_index("c")
    base = wid * B_PER_W
    pltpu.sync_copy(idx_hbm.at[pl.ds(wid * CHUNKS_PER_W, CHUNKS_PER_W)], idx_v)
    copies = []
    for j in range(CHUNKS_PER_W):
        copies.append(pltpu.async_copy(
            tab_hbm.at[idx_v.at[j]], rows_v.at[pl.ds(j * CHUNK, CHUNK)], sem))
    for cp in copies:
        cp.wait()
    pltpu.sync_copy(rows_v, out_hbm.at[pl.ds(base, B_PER_W)])


_sc_gather = functools.partial(
    pl.kernel,
    mesh=plsc.VectorSubcoreMesh(
        core_axis_name="c", subcore_axis_name="s",
        num_cores=NUM_CORES, num_subcores=NUM_SUBCORES),
    out_type=jax.ShapeDtypeStruct((BATCH, ROW_PAD), jnp.float32),
    scratch_types=[
        pltpu.VMEM((CHUNKS_PER_W, CHUNK), jnp.int32),
        pltpu.VMEM((B_PER_W, ROW_PAD), jnp.float32),
        pltpu.SemaphoreType.DMA,
    ],
)(_sc_gather_body)


def _mlp_body(x1_ref, cidx_ref, ctab_ref, w1a_ref, w1b_ref, b1_ref, w2_ref,
              b2_ref, o_ref):
    m = x1_ref.shape[0]
    x1 = x1_ref[:, :EMBED_DIM]
    onehot = (jax.lax.broadcasted_iota(jnp.int32, (m, CAT_PAD), 1)
              == cidx_ref[...]).astype(jnp.float32)
    x2 = jnp.dot(onehot, ctab_ref[...], preferred_element_type=jnp.float32)
    h = jnp.dot(x1, w1a_ref[...], preferred_element_type=jnp.float32)
    h = h + jnp.dot(x2, w1b_ref[...], preferred_element_type=jnp.float32)
    h = jnp.maximum(h + b1_ref[...], 0.0)
    out = jnp.dot(h, w2_ref[...], preferred_element_type=jnp.float32) + b2_ref[...]
    norm = jnp.sqrt(jnp.sum(out * out, axis=1, keepdims=True))
    o_ref[...] = out / jnp.maximum(norm, 1e-12)


def _mlp(trows, cidx, ctab_pad, w1a, w1b, b1, w2, b2, block_m=2048):
    grid = (BATCH // block_m,)
    return pl.pallas_call(
        _mlp_body,
        grid=grid,
        in_specs=[
            pl.BlockSpec((block_m, ROW_PAD), lambda i: (i, 0)),
            pl.BlockSpec((block_m, 1), lambda i: (i, 0)),
            pl.BlockSpec((CAT_PAD, EMBED_DIM), lambda i: (0, 0)),
            pl.BlockSpec((EMBED_DIM, HIDDEN_DIM), lambda i: (0, 0)),
            pl.BlockSpec((EMBED_DIM, HIDDEN_DIM), lambda i: (0, 0)),
            pl.BlockSpec((1, HIDDEN_DIM), lambda i: (0, 0)),
            pl.BlockSpec((HIDDEN_DIM, EMBED_DIM), lambda i: (0, 0)),
            pl.BlockSpec((1, EMBED_DIM), lambda i: (0, 0)),
        ],
        out_specs=pl.BlockSpec((block_m, EMBED_DIM), lambda i: (i, 0)),
        out_shape=jax.ShapeDtypeStruct((BATCH, EMBED_DIM), jnp.float32),
    )(trows, cidx, ctab_pad, w1a, w1b, b1, w2, b2)


def kernel(title_idx, category_idx, title_table, category_table, W1, b1, W2, b2):
    tidx = title_idx.astype(jnp.int32).reshape(BATCH // CHUNK, CHUNK)
    eye = jnp.eye(EMBED_DIM, dtype=jnp.float32)
    tab2 = _transpose_table(title_table.T, eye)
    trows = _sc_gather(tidx, tab2)
    ctab_pad = jnp.zeros((CAT_PAD, EMBED_DIM), jnp.float32).at[:CATEGORY_SIZE].set(
        category_table)
    w1t = W1.T
    return _mlp(trows, category_idx.astype(jnp.int32).reshape(BATCH, 1),
                ctab_pad, w1t[:EMBED_DIM], w1t[EMBED_DIM:],
                b1.reshape(1, HIDDEN_DIM), W2.T, b2.reshape(1, EMBED_DIM))


# speedup vs baseline: 26.8003x; 1.0561x over previous
import functools

import jax
import jax.numpy as jnp
from jax import lax
from jax.experimental import pallas as pl
from jax.experimental.pallas import tpu as pltpu
from jax.experimental.pallas import tpu_sc as plsc

EMBED_DIM = 64
ROW_PAD = 128
HIDDEN_DIM = 256
VOCAB_SIZE = 1000000
CATEGORY_SIZE = 1000
CAT_PAD = 1024
BATCH = 16384

NUM_CORES = 2
NUM_SUBCORES = 16
NUM_WORKERS = NUM_CORES * NUM_SUBCORES
B_PER_W = BATCH // NUM_WORKERS
CHUNK = 128
CHUNKS_PER_W = B_PER_W // CHUNK

TBLK = 16384
THALF = TBLK // 2
TGRID = -(-VOCAB_SIZE // TBLK)
VOCAB_PACK = TGRID * THALF


def _transpose_body(x_ref, eye_ref, o_ref):
    xt = jax.lax.dot_general(x_ref[...], eye_ref[...], (((0,), (0,)), ((), ())),
                             preferred_element_type=jnp.float32)
    o_ref[:, :EMBED_DIM] = xt[:THALF]
    o_ref[:, EMBED_DIM:] = xt[THALF:]


def _transpose_table(tabt, eye):
    return pl.pallas_call(
        _transpose_body,
        grid=(TGRID,),
        in_specs=[
            pl.BlockSpec((EMBED_DIM, TBLK), lambda i: (0, i)),
            pl.BlockSpec((EMBED_DIM, EMBED_DIM), lambda i: (0, 0)),
        ],
        out_specs=pl.BlockSpec((THALF, ROW_PAD), lambda i: (i, 0)),
        out_shape=jax.ShapeDtypeStruct((VOCAB_PACK, ROW_PAD), jnp.float32),
    )(tabt, eye)


def _sc_gather_body(idx_hbm, tab_hbm, out_hbm, idx_v, rows_v, sem):
    wid = lax.axis_index("s") * NUM_CORES + lax.axis_index("c")
    base = wid * B_PER_W
    pltpu.sync_copy(idx_hbm.at[pl.ds(wid * CHUNKS_PER_W, CHUNKS_PER_W)], idx_v)
    for g in range(CHUNKS_PER_W):
        for k in range(CHUNK // 16):
            v = idx_v[g, pl.ds(k * 16, 16)]
            idx_v[g, pl.ds(k * 16, 16)] = (
                jax.lax.shift_left(jax.lax.shift_right_logical(v, 14), 13)
                + jax.lax.bitwise_and(v, THALF - 1))
    copies = []
    for j in range(CHUNKS_PER_W):
        copies.append(pltpu.async_copy(
            tab_hbm.at[idx_v.at[j]], rows_v.at[pl.ds(j * CHUNK, CHUNK)], sem))
    for cp in copies:
        cp.wait()
    pltpu.sync_copy(rows_v, out_hbm.at[pl.ds(base, B_PER_W)])


_sc_gather = functools.partial(
    pl.kernel,
    mesh=plsc.VectorSubcoreMesh(
        core_axis_name="c", subcore_axis_name="s",
        num_cores=NUM_CORES, num_subcores=NUM_SUBCORES),
    out_type=jax.ShapeDtypeStruct((BATCH, ROW_PAD), jnp.float32),
    scratch_types=[
        pltpu.VMEM((CHUNKS_PER_W, CHUNK), jnp.int32),
        pltpu.VMEM((B_PER_W, ROW_PAD), jnp.float32),
        pltpu.SemaphoreType.DMA,
    ],
)(_sc_gather_body)


def _mlp_body(x1_ref, tidx_ref, cidx_ref, ctab_ref, w1a_ref, w1b_ref, b1_ref,
              w2_ref, b2_ref, o_ref):
    m = x1_ref.shape[0]
    half = jax.lax.bitwise_and(
        jax.lax.shift_right_logical(tidx_ref[...], 13), 1)
    x1 = jnp.where(half == 1, x1_ref[:, EMBED_DIM:], x1_ref[:, :EMBED_DIM])
    onehot = (jax.lax.broadcasted_iota(jnp.int32, (m, CAT_PAD), 1)
              == cidx_ref[...]).astype(jnp.float32)
    x2 = jnp.dot(onehot, ctab_ref[...], preferred_element_type=jnp.float32)
    h = jnp.dot(x1, w1a_ref[...], preferred_element_type=jnp.float32)
    h = h + jnp.dot(x2, w1b_ref[...], preferred_element_type=jnp.float32)
    h = jnp.maximum(h + b1_ref[...], 0.0)
    out = jnp.dot(h, w2_ref[...], preferred_element_type=jnp.float32) + b2_ref[...]
    norm = jnp.sqrt(jnp.sum(out * out, axis=1, keepdims=True))
    o_ref[...] = out / jnp.maximum(norm, 1e-12)


def _mlp(trows, tidx, cidx, ctab_pad, w1a, w1b, b1, w2, b2, block_m=2048):
    grid = (BATCH // block_m,)
    return pl.pallas_call(
        _mlp_body,
        grid=grid,
        in_specs=[
            pl.BlockSpec((block_m, ROW_PAD), lambda i: (i, 0)),
            pl.BlockSpec((block_m, 1), lambda i: (i, 0)),
            pl.BlockSpec((block_m, 1), lambda i: (i, 0)),
            pl.BlockSpec((CAT_PAD, EMBED_DIM), lambda i: (0, 0)),
            pl.BlockSpec((EMBED_DIM, HIDDEN_DIM), lambda i: (0, 0)),
            pl.BlockSpec((EMBED_DIM, HIDDEN_DIM), lambda i: (0, 0)),
            pl.BlockSpec((1, HIDDEN_DIM), lambda i: (0, 0)),
            pl.BlockSpec((HIDDEN_DIM, EMBED_DIM), lambda i: (0, 0)),
            pl.BlockSpec((1, EMBED_DIM), lambda i: (0, 0)),
        ],
        out_specs=pl.BlockSpec((block_m, EMBED_DIM), lambda i: (i, 0)),
        out_shape=jax.ShapeDtypeStruct((BATCH, EMBED_DIM), jnp.float32),
    )(trows, tidx, cidx, ctab_pad, w1a, w1b, b1, w2, b2)


def kernel(title_idx, category_idx, title_table, category_table, W1, b1, W2, b2):
    tidx = title_idx.astype(jnp.int32)
    tidx2 = tidx.reshape(BATCH // CHUNK, CHUNK)
    eye = jnp.eye(EMBED_DIM, dtype=jnp.float32)
    tab2 = _transpose_table(title_table.T, eye)
    trows = _sc_gather(tidx2, tab2)
    ctab_pad = jnp.zeros((CAT_PAD, EMBED_DIM), jnp.float32).at[:CATEGORY_SIZE].set(
        category_table)
    w1t = W1.T
    return _mlp(trows, tidx.reshape(BATCH, 1),
                category_idx.astype(jnp.int32).reshape(BATCH, 1),
                ctab_pad, w1t[:EMBED_DIM], w1t[EMBED_DIM:],
                b1.reshape(1, HIDDEN_DIM), W2.T, b2.reshape(1, EMBED_DIM))


# speedup vs baseline: 32.2163x; 1.2021x over previous
import functools

import jax
import jax.numpy as jnp
from jax import lax
from jax.experimental import pallas as pl
from jax.experimental.pallas import tpu as pltpu
from jax.experimental.pallas import tpu_sc as plsc

EMBED_DIM = 64
ROW_PAD = 128
HIDDEN_DIM = 256
VOCAB_SIZE = 1000000
CATEGORY_SIZE = 1000
CAT_PAD = 1024
BATCH = 16384

NUM_CORES = 2
NUM_SUBCORES = 16
NUM_WORKERS = NUM_CORES * NUM_SUBCORES
B_PER_W = BATCH // NUM_WORKERS
CHUNK = 128
CHUNKS_PER_W = B_PER_W // CHUNK

TBLK = 16384
TIN = 2 * TBLK
TGRID = -(-VOCAB_SIZE // TIN)
VOCAB_PACK = TGRID * TBLK


def _transpose_body(x_ref, ea_ref, eb_ref, o_ref):
    dn = (((0,), (0,)), ((), ()))
    o_ref[...] = (
        jax.lax.dot_general(x_ref[:, :TBLK], ea_ref[...], dn,
                            preferred_element_type=jnp.float32)
        + jax.lax.dot_general(x_ref[:, TBLK:], eb_ref[...], dn,
                              preferred_element_type=jnp.float32))


def _transpose_table(tabt, ea, eb):
    return pl.pallas_call(
        _transpose_body,
        grid=(TGRID,),
        in_specs=[
            pl.BlockSpec((EMBED_DIM, TIN), lambda i: (0, i)),
            pl.BlockSpec((EMBED_DIM, ROW_PAD), lambda i: (0, 0)),
            pl.BlockSpec((EMBED_DIM, ROW_PAD), lambda i: (0, 0)),
        ],
        out_specs=pl.BlockSpec((TBLK, ROW_PAD), lambda i: (i, 0)),
        out_shape=jax.ShapeDtypeStruct((VOCAB_PACK, ROW_PAD), jnp.float32),
    )(tabt, ea, eb)


def _sc_gather_body(idx_hbm, tab_hbm, out_hbm, idx_v, rows_v, sem):
    wid = lax.axis_index("s") * NUM_CORES + lax.axis_index("c")
    base = wid * B_PER_W
    pltpu.sync_copy(idx_hbm.at[pl.ds(wid * CHUNKS_PER_W, CHUNKS_PER_W)], idx_v)
    for g in range(CHUNKS_PER_W):
        for k in range(CHUNK // 16):
            v = idx_v[g, pl.ds(k * 16, 16)]
            idx_v[g, pl.ds(k * 16, 16)] = (
                jax.lax.shift_left(jax.lax.shift_right_logical(v, 15), 14)
                + jax.lax.bitwise_and(v, TBLK - 1))
    copies = []
    for j in range(CHUNKS_PER_W):
        copies.append(pltpu.async_copy(
            tab_hbm.at[idx_v.at[j]], rows_v.at[pl.ds(j * CHUNK, CHUNK)], sem))
    for cp in copies:
        cp.wait()
    pltpu.sync_copy(rows_v, out_hbm.at[pl.ds(base, B_PER_W)])


_sc_gather = functools.partial(
    pl.kernel,
    mesh=plsc.VectorSubcoreMesh(
        core_axis_name="c", subcore_axis_name="s",
        num_cores=NUM_CORES, num_subcores=NUM_SUBCORES),
    out_type=jax.ShapeDtypeStruct((BATCH, ROW_PAD), jnp.float32),
    scratch_types=[
        pltpu.VMEM((CHUNKS_PER_W, CHUNK), jnp.int32),
        pltpu.VMEM((B_PER_W, ROW_PAD), jnp.float32),
        pltpu.SemaphoreType.DMA,
    ],
)(_sc_gather_body)


def _mlp_body(x1_ref, tidx_ref, cidx_ref, ctab_ref, w1a_ref, w1b_ref, b1_ref,
              w2_ref, b2_ref, o_ref):
    m = x1_ref.shape[0]
    half = jax.lax.bitwise_and(
        jax.lax.shift_right_logical(tidx_ref[...], 14), 1)
    x1 = jnp.where(half == 1, x1_ref[:, EMBED_DIM:], x1_ref[:, :EMBED_DIM])
    onehot = (jax.lax.broadcasted_iota(jnp.int32, (m, CAT_PAD), 1)
              == cidx_ref[...]).astype(jnp.float32)
    x2 = jnp.dot(onehot, ctab_ref[...], preferred_element_type=jnp.float32)
    h = jnp.dot(x1, w1a_ref[...], preferred_element_type=jnp.float32)
    h = h + jnp.dot(x2, w1b_ref[...], preferred_element_type=jnp.float32)
    h = jnp.maximum(h + b1_ref[...], 0.0)
    out = jnp.dot(h, w2_ref[...], preferred_element_type=jnp.float32) + b2_ref[...]
    norm = jnp.sqrt(jnp.sum(out * out, axis=1, keepdims=True))
    o_ref[...] = out / jnp.maximum(norm, 1e-12)


def _mlp(trows, tidx, cidx, ctab_pad, w1a, w1b, b1, w2, b2, block_m=2048):
    grid = (BATCH // block_m,)
    return pl.pallas_call(
        _mlp_body,
        grid=grid,
        in_specs=[
            pl.BlockSpec((block_m, ROW_PAD), lambda i: (i, 0)),
            pl.BlockSpec((block_m, 1), lambda i: (i, 0)),
            pl.BlockSpec((block_m, 1), lambda i: (i, 0)),
            pl.BlockSpec((CAT_PAD, EMBED_DIM), lambda i: (0, 0)),
            pl.BlockSpec((EMBED_DIM, HIDDEN_DIM), lambda i: (0, 0)),
            pl.BlockSpec((EMBED_DIM, HIDDEN_DIM), lambda i: (0, 0)),
            pl.BlockSpec((1, HIDDEN_DIM), lambda i: (0, 0)),
            pl.BlockSpec((HIDDEN_DIM, EMBED_DIM), lambda i: (0, 0)),
            pl.BlockSpec((1, EMBED_DIM), lambda i: (0, 0)),
        ],
        out_specs=pl.BlockSpec((block_m, EMBED_DIM), lambda i: (i, 0)),
        out_shape=jax.ShapeDtypeStruct((BATCH, EMBED_DIM), jnp.float32),
    )(trows, tidx, cidx, ctab_pad, w1a, w1b, b1, w2, b2)


def kernel(title_idx, category_idx, title_table, category_table, W1, b1, W2, b2):
    tidx = title_idx.astype(jnp.int32)
    tidx2 = tidx.reshape(BATCH // CHUNK, CHUNK)
    eye = jnp.eye(EMBED_DIM, dtype=jnp.float32)
    zero = jnp.zeros((EMBED_DIM, EMBED_DIM), jnp.float32)
    ea = jnp.concatenate([eye, zero], axis=1)
    eb = jnp.concatenate([zero, eye], axis=1)
    tab2 = _transpose_table(title_table.T, ea, eb)
    trows = _sc_gather(tidx2, tab2)
    ctab_pad = jnp.zeros((CAT_PAD, EMBED_DIM), jnp.float32).at[:CATEGORY_SIZE].set(
        category_table)
    w1t = W1.T
    return _mlp(trows, tidx.reshape(BATCH, 1),
                category_idx.astype(jnp.int32).reshape(BATCH, 1),
                ctab_pad, w1t[:EMBED_DIM], w1t[EMBED_DIM:],
                b1.reshape(1, HIDDEN_DIM), W2.T, b2.reshape(1, EMBED_DIM))


# speedup vs baseline: 32.2302x; 1.0004x over previous
import functools

import jax
import jax.numpy as jnp
from jax import lax
from jax.experimental import pallas as pl
from jax.experimental.pallas import tpu as pltpu
from jax.experimental.pallas import tpu_sc as plsc

EMBED_DIM = 64
ROW_PAD = 128
HIDDEN_DIM = 256
VOCAB_SIZE = 1000000
CATEGORY_SIZE = 1000
CAT_PAD = 1024
BATCH = 16384

NUM_CORES = 2
NUM_SUBCORES = 16
NUM_WORKERS = NUM_CORES * NUM_SUBCORES
B_PER_W = BATCH // NUM_WORKERS
CHUNK = 128
CHUNKS_PER_W = B_PER_W // CHUNK

TBLK = 16384
TIN = 2 * TBLK
TGRID = -(-VOCAB_SIZE // TIN)
VOCAB_PACK = TGRID * TBLK


def _transpose_body(x_ref, ea_ref, eb_ref, o_ref):
    dn = (((0,), (0,)), ((), ()))
    o_ref[...] = (
        jax.lax.dot_general(x_ref[:, :TBLK], ea_ref[...], dn,
                            preferred_element_type=jnp.float32)
        + jax.lax.dot_general(x_ref[:, TBLK:], eb_ref[...], dn,
                              preferred_element_type=jnp.float32))


def _transpose_table(tabt, ea, eb):
    return pl.pallas_call(
        _transpose_body,
        grid=(TGRID,),
        in_specs=[
            pl.BlockSpec((EMBED_DIM, TIN), lambda i: (0, i)),
            pl.BlockSpec((EMBED_DIM, ROW_PAD), lambda i: (0, 0)),
            pl.BlockSpec((EMBED_DIM, ROW_PAD), lambda i: (0, 0)),
        ],
        out_specs=pl.BlockSpec((TBLK, ROW_PAD), lambda i: (i, 0)),
        out_shape=jax.ShapeDtypeStruct((VOCAB_PACK, ROW_PAD), jnp.float32),
    )(tabt, ea, eb)


def _sc_gather_body(idx_hbm, tab_hbm, out_hbm, idx_v, rows_v, sem):
    wid = lax.axis_index("s") * NUM_CORES + lax.axis_index("c")
    base = wid * B_PER_W
    pltpu.sync_copy(idx_hbm.at[pl.ds(wid * CHUNKS_PER_W, CHUNKS_PER_W)], idx_v)
    for g in range(CHUNKS_PER_W):
        for k in range(CHUNK // 16):
            v = idx_v[g, pl.ds(k * 16, 16)]
            idx_v[g, pl.ds(k * 16, 16)] = (
                jax.lax.shift_left(jax.lax.shift_right_logical(v, 15), 14)
                + jax.lax.bitwise_and(v, TBLK - 1))
    copies = []
    for j in range(CHUNKS_PER_W):
        copies.append(pltpu.async_copy(
            tab_hbm.at[idx_v.at[j]], rows_v.at[pl.ds(j * CHUNK, CHUNK)], sem))
    for cp in copies:
        cp.wait()
    pltpu.sync_copy(rows_v, out_hbm.at[pl.ds(base, B_PER_W)])


_sc_gather = functools.partial(
    pl.kernel,
    mesh=plsc.VectorSubcoreMesh(
        core_axis_name="c", subcore_axis_name="s",
        num_cores=NUM_CORES, num_subcores=NUM_SUBCORES),
    out_type=jax.ShapeDtypeStruct((BATCH, ROW_PAD), jnp.float32),
    scratch_types=[
        pltpu.VMEM((CHUNKS_PER_W, CHUNK), jnp.int32),
        pltpu.VMEM((B_PER_W, ROW_PAD), jnp.float32),
        pltpu.SemaphoreType.DMA,
    ],
)(_sc_gather_body)


def _catvec_body(cidx_ref, ctab_ref, o_ref):
    m = cidx_ref.shape[0]
    onehot = (jax.lax.broadcasted_iota(jnp.int32, (m, CAT_PAD), 1)
              == cidx_ref[...]).astype(jnp.float32)
    o_ref[...] = jnp.dot(onehot, ctab_ref[...],
                         preferred_element_type=jnp.float32)


def _catvec(cidx, ctab_pad, block_m=4096):
    return pl.pallas_call(
        _catvec_body,
        grid=(BATCH // block_m,),
        in_specs=[
            pl.BlockSpec((block_m, 1), lambda i: (i, 0)),
            pl.BlockSpec((CAT_PAD, EMBED_DIM), lambda i: (0, 0)),
        ],
        out_specs=pl.BlockSpec((block_m, EMBED_DIM), lambda i: (i, 0)),
        out_shape=jax.ShapeDtypeStruct((BATCH, EMBED_DIM), jnp.float32),
    )(cidx, ctab_pad)


def _mlp_body(x1_ref, tidx_ref, x2_ref, w1a_ref, w1b_ref, b1_ref,
              w2_ref, b2_ref, o_ref):
    half = jax.lax.bitwise_and(
        jax.lax.shift_right_logical(tidx_ref[...], 14), 1)
    x1 = jnp.where(half == 1, x1_ref[:, EMBED_DIM:], x1_ref[:, :EMBED_DIM])
    x2 = x2_ref[...]
    h = jnp.dot(x1, w1a_ref[...], preferred_element_type=jnp.float32)
    h = h + jnp.dot(x2, w1b_ref[...], preferred_element_type=jnp.float32)
    h = jnp.maximum(h + b1_ref[...], 0.0)
    out = jnp.dot(h, w2_ref[...], preferred_element_type=jnp.float32) + b2_ref[...]
    norm = jnp.sqrt(jnp.sum(out * out, axis=1, keepdims=True))
    o_ref[...] = out / jnp.maximum(norm, 1e-12)


def _mlp(trows, tidx, catvec, w1a, w1b, b1, w2, b2, block_m=2048):
    grid = (BATCH // block_m,)
    return pl.pallas_call(
        _mlp_body,
        grid=grid,
        in_specs=[
            pl.BlockSpec((block_m, ROW_PAD), lambda i: (i, 0)),
            pl.BlockSpec((block_m, 1), lambda i: (i, 0)),
            pl.BlockSpec((block_m, EMBED_DIM), lambda i: (i, 0)),
            pl.BlockSpec((EMBED_DIM, HIDDEN_DIM), lambda i: (0, 0)),
            pl.BlockSpec((EMBED_DIM, HIDDEN_DIM), lambda i: (0, 0)),
            pl.BlockSpec((1, HIDDEN_DIM), lambda i: (0, 0)),
            pl.BlockSpec((HIDDEN_DIM, EMBED_DIM), lambda i: (0, 0)),
            pl.BlockSpec((1, EMBED_DIM), lambda i: (0, 0)),
        ],
        out_specs=pl.BlockSpec((block_m, EMBED_DIM), lambda i: (i, 0)),
        out_shape=jax.ShapeDtypeStruct((BATCH, EMBED_DIM), jnp.float32),
    )(trows, tidx, catvec, w1a, w1b, b1, w2, b2)


def kernel(title_idx, category_idx, title_table, category_table, W1, b1, W2, b2):
    tidx = title_idx.astype(jnp.int32)
    tidx2 = tidx.reshape(BATCH // CHUNK, CHUNK)
    eye = jnp.eye(EMBED_DIM, dtype=jnp.float32)
    zero = jnp.zeros((EMBED_DIM, EMBED_DIM), jnp.float32)
    ea = jnp.concatenate([eye, zero], axis=1)
    eb = jnp.concatenate([zero, eye], axis=1)
    tab2 = _transpose_table(title_table.T, ea, eb)
    trows = _sc_gather(tidx2, tab2)
    ctab_pad = jnp.zeros((CAT_PAD, EMBED_DIM), jnp.float32).at[:CATEGORY_SIZE].set(
        category_table)
    catvec = _catvec(category_idx.astype(jnp.int32).reshape(BATCH, 1), ctab_pad)
    w1t = W1.T
    return _mlp(trows, tidx.reshape(BATCH, 1), catvec,
                w1t[:EMBED_DIM], w1t[EMBED_DIM:],
                b1.reshape(1, HIDDEN_DIM), W2.T, b2.reshape(1, EMBED_DIM))


# speedup vs baseline: 32.4238x; 1.0060x over previous
import functools

import jax
import jax.numpy as jnp
from jax import lax
from jax.experimental import pallas as pl
from jax.experimental.pallas import tpu as pltpu
from jax.experimental.pallas import tpu_sc as plsc

EMBED_DIM = 64
ROW_PAD = 128
HIDDEN_DIM = 256
VOCAB_SIZE = 1000000
CATEGORY_SIZE = 1000
CAT_PAD = 1024
BATCH = 16384

NUM_CORES = 2
NUM_SUBCORES = 16
NUM_WORKERS = NUM_CORES * NUM_SUBCORES
B_PER_W = BATCH // NUM_WORKERS
CHUNK = 128
CHUNKS_PER_W = B_PER_W // CHUNK

TBLK = 16384
TIN = 2 * TBLK
TGRID = -(-VOCAB_SIZE // TIN)
VOCAB_PACK = TGRID * TBLK


def _transpose_body(x_ref, ea_ref, eb_ref, o_ref):
    dn = (((0,), (0,)), ((), ()))
    o_ref[...] = (
        jax.lax.dot_general(x_ref[:, :TBLK], ea_ref[...], dn,
                            preferred_element_type=jnp.float32)
        + jax.lax.dot_general(x_ref[:, TBLK:], eb_ref[...], dn,
                              preferred_element_type=jnp.float32))


def _transpose_table(tabt, ea, eb):
    return pl.pallas_call(
        _transpose_body,
        grid=(TGRID,),
        in_specs=[
            pl.BlockSpec((EMBED_DIM, TIN), lambda i: (0, i)),
            pl.BlockSpec((EMBED_DIM, ROW_PAD), lambda i: (0, 0)),
            pl.BlockSpec((EMBED_DIM, ROW_PAD), lambda i: (0, 0)),
        ],
        out_specs=pl.BlockSpec((TBLK, ROW_PAD), lambda i: (i, 0)),
        out_shape=jax.ShapeDtypeStruct((VOCAB_PACK, ROW_PAD), jnp.float32),
    )(tabt, ea, eb)


def _sc_gather_body(idx_hbm, tab_hbm, out_hbm, idx_v, rows_v, sem):
    wid = lax.axis_index("s") * NUM_CORES + lax.axis_index("c")
    base = wid * B_PER_W
    pltpu.sync_copy(idx_hbm.at[pl.ds(wid * CHUNKS_PER_W, CHUNKS_PER_W)], idx_v)
    for g in range(CHUNKS_PER_W):
        for k in range(CHUNK // 16):
            v = idx_v[g, pl.ds(k * 16, 16)]
            idx_v[g, pl.ds(k * 16, 16)] = (
                jax.lax.shift_left(jax.lax.shift_right_logical(v, 15), 14)
                + jax.lax.bitwise_and(v, TBLK - 1))
    copies = []
    for j in range(CHUNKS_PER_W):
        copies.append(pltpu.async_copy(
            tab_hbm.at[idx_v.at[j]], rows_v.at[pl.ds(j * CHUNK, CHUNK)], sem))
    for cp in copies:
        cp.wait()
    pltpu.sync_copy(rows_v, out_hbm.at[pl.ds(base, B_PER_W)])


_sc_gather = functools.partial(
    pl.kernel,
    mesh=plsc.VectorSubcoreMesh(
        core_axis_name="c", subcore_axis_name="s",
        num_cores=NUM_CORES, num_subcores=NUM_SUBCORES),
    out_type=jax.ShapeDtypeStruct((BATCH, ROW_PAD), jnp.float32),
    scratch_types=[
        pltpu.VMEM((CHUNKS_PER_W, CHUNK), jnp.int32),
        pltpu.VMEM((B_PER_W, ROW_PAD), jnp.float32),
        pltpu.SemaphoreType.DMA,
    ],
)(_sc_gather_body)


def _catvec_body(cidx_ref, ctab_ref, o_ref):
    m = cidx_ref.shape[0]
    onehot = (jax.lax.broadcasted_iota(jnp.int32, (m, CAT_PAD), 1)
              == cidx_ref[...]).astype(jnp.float32)
    o_ref[...] = jnp.dot(onehot, ctab_ref[...],
                         preferred_element_type=jnp.float32)


def _catvec(cidx, ctab_pad, block_m=4096):
    return pl.pallas_call(
        _catvec_body,
        grid=(BATCH // block_m,),
        in_specs=[
            pl.BlockSpec((block_m, 1), lambda i: (i, 0)),
            pl.BlockSpec((CAT_PAD, EMBED_DIM), lambda i: (0, 0)),
        ],
        out_specs=pl.BlockSpec((block_m, EMBED_DIM), lambda i: (i, 0)),
        out_shape=jax.ShapeDtypeStruct((BATCH, EMBED_DIM), jnp.float32),
    )(cidx, ctab_pad)


def _mlp_body(x1_ref, tidx_ref, x2_ref, w1a_ref, w1b_ref, b1_ref,
              w2_ref, b2_ref, o_ref):
    half = jax.lax.bitwise_and(
        jax.lax.shift_right_logical(tidx_ref[...], 14), 1)
    x1 = jnp.where(half == 1, x1_ref[:, EMBED_DIM:], x1_ref[:, :EMBED_DIM])
    x2 = x2_ref[...]
    h = jnp.dot(x1, w1a_ref[...], preferred_element_type=jnp.float32)
    h = h + jnp.dot(x2, w1b_ref[...], preferred_element_type=jnp.float32)
    h = jnp.maximum(h + b1_ref[...], 0.0)
    out = jnp.dot(h, w2_ref[...], preferred_element_type=jnp.float32) + b2_ref[...]
    norm = jnp.sqrt(jnp.sum(out * out, axis=1, keepdims=True))
    o_ref[...] = out / jnp.maximum(norm, 1e-12)


def _mlp(trows, tidx, catvec, w1a, w1b, b1, w2, b2, block_m=4096):
    grid = (BATCH // block_m,)
    return pl.pallas_call(
        _mlp_body,
        grid=grid,
        in_specs=[
            pl.BlockSpec((block_m, ROW_PAD), lambda i: (i, 0)),
            pl.BlockSpec((block_m, 1), lambda i: (i, 0)),
            pl.BlockSpec((block_m, EMBED_DIM), lambda i: (i, 0)),
            pl.BlockSpec((EMBED_DIM, HIDDEN_DIM), lambda i: (0, 0)),
            pl.BlockSpec((EMBED_DIM, HIDDEN_DIM), lambda i: (0, 0)),
            pl.BlockSpec((1, HIDDEN_DIM), lambda i: (0, 0)),
            pl.BlockSpec((HIDDEN_DIM, EMBED_DIM), lambda i: (0, 0)),
            pl.BlockSpec((1, EMBED_DIM), lambda i: (0, 0)),
        ],
        out_specs=pl.BlockSpec((block_m, EMBED_DIM), lambda i: (i, 0)),
        out_shape=jax.ShapeDtypeStruct((BATCH, EMBED_DIM), jnp.float32),
    )(trows, tidx, catvec, w1a, w1b, b1, w2, b2)


def kernel(title_idx, category_idx, title_table, category_table, W1, b1, W2, b2):
    tidx = title_idx.astype(jnp.int32)
    tidx2 = tidx.reshape(BATCH // CHUNK, CHUNK)
    eye = jnp.eye(EMBED_DIM, dtype=jnp.float32)
    zero = jnp.zeros((EMBED_DIM, EMBED_DIM), jnp.float32)
    ea = jnp.concatenate([eye, zero], axis=1)
    eb = jnp.concatenate([zero, eye], axis=1)
    tab2 = _transpose_table(title_table.T, ea, eb)
    trows = _sc_gather(tidx2, tab2)
    ctab_pad = jnp.zeros((CAT_PAD, EMBED_DIM), jnp.float32).at[:CATEGORY_SIZE].set(
        category_table)
    catvec = _catvec(category_idx.astype(jnp.int32).reshape(BATCH, 1), ctab_pad)
    w1t = W1.T
    return _mlp(trows, tidx.reshape(BATCH, 1), catvec,
                w1t[:EMBED_DIM], w1t[EMBED_DIM:],
                b1.reshape(1, HIDDEN_DIM), W2.T, b2.reshape(1, EMBED_DIM))
